# baseline (speedup 1.0000x reference)
import functools

import jax
import jax.numpy as jnp
from jax import lax
from jax.experimental import pallas as pl
from jax.experimental.pallas import tpu as pltpu
from jax.experimental.pallas import tpu_sc as plsc

_BM = 512


def _ln_argmax_body(x_ref, wt_ref, g_ref, b_ref, iota_ref, idx_ref,
                    logits_ref):
    i = pl.program_id(0)
    k = wt_ref.shape[1]
    nrg = _BM // 128
    nch = k // 128

    def pipeline_step(dot_ref, scan_ref):
        x = x_ref[...]
        mu = jnp.mean(x, axis=-1, keepdims=True)
        var = jnp.mean((x - mu) ** 2, axis=-1, keepdims=True)
        h = (x - mu) / jnp.sqrt(var + 1e-5) * g_ref[...] + b_ref[...]
        dot_ref[...] = jnp.dot(h, wt_ref[...],
                               preferred_element_type=jnp.float32)
        lane_f = iota_ref[0:1, 0:128]
        big = jnp.float32(3.0e38)
        for g in range(nrg):
            m_run = scan_ref[pl.ds(g * 128, 128), pl.ds(0, 128)]
            c_run = jnp.zeros((128, 128), jnp.float32)
            for c in range(1, nch):
                chunk = scan_ref[pl.ds(g * 128, 128), pl.ds(c * 128, 128)]
                gt = chunk > m_run
                m_run = jnp.where(gt, chunk, m_run)
                c_run = jnp.where(gt, jnp.float32(c), c_run)
            m = jnp.max(m_run, axis=-1, keepdims=True)
            cand = jnp.where(m_run == m, c_run * 128.0 + lane_f, big)
            idxf = jnp.min(cand, axis=-1)
            idx_ref[0, 0, pl.ds(g * 128, 128)] = idxf.astype(jnp.int32)

    @pl.when(i % 2 == 0)
    def _():
        pipeline_step(logits_ref.at[0], logits_ref.at[1])

    @pl.when(i % 2 == 1)
    def _():
        pipeline_step(logits_ref.at[1], logits_ref.at[0])


def _ln_argmax(x2, wt, g2, b2, iota_f):
    n, d = x2.shape
    k = wt.shape[1]
    nblk = n // _BM
    grid = nblk + 1
    idx3 = pl.pallas_call(
        _ln_argmax_body,
        grid=(grid,),
        in_specs=[
            pl.BlockSpec((_BM, d), lambda i: (jnp.minimum(i, nblk - 1), 0)),
            pl.BlockSpec((d, k), lambda i: (0, 0)),
            pl.BlockSpec((1, d), lambda i: (0, 0)),
            pl.BlockSpec((1, d), lambda i: (0, 0)),
            pl.BlockSpec((1, k), lambda i: (0, 0)),
        ],
        out_specs=pl.BlockSpec(
            (1, 1, _BM), lambda i: (jnp.maximum(i, 1) - 1, 0, 0)),
        out_shape=jax.ShapeDtypeStruct((nblk, 1, _BM), jnp.int32),
        scratch_shapes=[pltpu.VMEM((2, _BM, k), jnp.float32)],
    )(x2, wt, g2, b2, iota_f)
    return idx3.reshape(n)


def _sc_gather(table, idx):
    b = idx.shape[0]
    d = table.shape[1]
    nw = 32
    b_per_w = b // nw
    mesh = plsc.VectorSubcoreMesh(core_axis_name="c", subcore_axis_name="s")

    @functools.partial(
        pl.kernel,
        mesh=mesh,
        compiler_params=pltpu.CompilerParams(use_tc_tiling_on_sc=False),
        out_type=jax.ShapeDtypeStruct((b, d), jnp.float32),
        scratch_types=[
            pltpu.VMEM((b_per_w,), jnp.int32),
            pltpu.VMEM((b_per_w, d), jnp.float32),
            pltpu.SemaphoreType.DMA,
        ],
    )
    def k(table_hbm, idx_hbm, out_hbm, idx_v, rows_v, sem):
        wid = lax.axis_index("s") * 2 + lax.axis_index("c")
        base = wid * b_per_w
        pltpu.sync_copy(idx_hbm.at[pl.ds(base, b_per_w)], idx_v)
        pltpu.async_copy(table_hbm.at[idx_v], rows_v, sem).wait()
        pltpu.sync_copy(rows_v, out_hbm.at[pl.ds(base, b_per_w)])

    return k(table, idx)


def kernel(input, embd_weight, ln_gamma, ln_beta):
    bsz, seq, d = input.shape
    n = bsz * seq
    k = embd_weight.shape[0]
    x2 = input.reshape(n, d)
    wt = embd_weight.T
    g2 = ln_gamma.reshape(1, d)
    b2 = ln_beta.reshape(1, d)
    iota_f = jnp.arange(k, dtype=jnp.float32).reshape(1, k)
    idx = _ln_argmax(x2, wt, g2, b2, iota_f)
    q = _sc_gather(embd_weight, idx)
    return q.reshape(bsz, seq, d), idx.reshape(bsz, seq)


# speedup vs baseline: 1.3521x; 1.3521x over previous
import functools

import jax
import jax.numpy as jnp
from jax import lax
from jax.experimental import pallas as pl
from jax.experimental.pallas import tpu as pltpu
from jax.experimental.pallas import tpu_sc as plsc

_BM = 1024


def _ln_argmax_body(x_ref, wt_ref, g_ref, b_ref, iota_ref, idx_ref):
    x = x_ref[...]
    mu = jnp.mean(x, axis=-1, keepdims=True)
    var = jnp.mean((x - mu) ** 2, axis=-1, keepdims=True)
    h = (x - mu) / jnp.sqrt(var + 1e-5) * g_ref[...] + b_ref[...]
    logits = lax.dot_general(
        h, wt_ref[...], (((1,), (1,)), ((), ())),
        preferred_element_type=jnp.float32)
    k = wt_ref.shape[0]
    nrg = _BM // 128
    nch = k // 128
    lane_f = iota_ref[0:1, 0:128]
    big = jnp.float32(3.0e38)
    for g in range(nrg):
        rows = lax.slice(logits, (g * 128, 0), ((g + 1) * 128, k))
        m_run = lax.slice(rows, (0, 0), (128, 128))
        c_run = jnp.zeros((128, 128), jnp.float32)
        for c in range(1, nch):
            chunk = lax.slice(rows, (0, c * 128), (128, (c + 1) * 128))
            gt = chunk > m_run
            m_run = jnp.where(gt, chunk, m_run)
            c_run = jnp.where(gt, jnp.float32(c), c_run)
        m = jnp.max(m_run, axis=-1, keepdims=True)
        cand = jnp.where(m_run == m, c_run * 128.0 + lane_f, big)
        idxf = jnp.min(cand, axis=-1)
        idx_ref[g * (128 // 128):(g + 1) * (128 // 128), :] = (
            idxf.astype(jnp.int32).reshape(1, 128))


def _ln_argmax(x2, wt, g2, b2, iota_f):
    n, d = x2.shape
    k = wt.shape[0]
    grid = n // _BM
    rows = _BM // 128
    idx2 = pl.pallas_call(
        _ln_argmax_body,
        grid=(grid,),
        in_specs=[
            pl.BlockSpec((_BM, d), lambda i: (i, 0)),
            pl.BlockSpec((k, d), lambda i: (0, 0)),
            pl.BlockSpec((1, d), lambda i: (0, 0)),
            pl.BlockSpec((1, d), lambda i: (0, 0)),
            pl.BlockSpec((1, k), lambda i: (0, 0)),
        ],
        out_specs=pl.BlockSpec((rows, 128), lambda i: (i, 0)),
        out_shape=jax.ShapeDtypeStruct((grid * rows, 128), jnp.int32),
    )(x2, wt, g2, b2, iota_f)
    return idx2.reshape(n)


def _sc_gather(table, idx):
    b = idx.shape[0]
    d = table.shape[1]
    nw = 32
    b_per_w = b // nw
    mesh = plsc.VectorSubcoreMesh(core_axis_name="c", subcore_axis_name="s")

    @functools.partial(
        pl.kernel,
        mesh=mesh,
        compiler_params=pltpu.CompilerParams(use_tc_tiling_on_sc=False),
        out_type=jax.ShapeDtypeStruct((b, d), jnp.float32),
        scratch_types=[
            pltpu.VMEM((b_per_w,), jnp.int32),
            pltpu.VMEM((b_per_w, d), jnp.float32),
            pltpu.SemaphoreType.DMA,
        ],
    )
    def k(table_hbm, idx_hbm, out_hbm, idx_v, rows_v, sem):
        wid = lax.axis_index("s") * 2 + lax.axis_index("c")
        base = wid * b_per_w
        pltpu.sync_copy(idx_hbm.at[pl.ds(base, b_per_w)], idx_v)
        pltpu.async_copy(table_hbm.at[idx_v], rows_v, sem).wait()
        pltpu.sync_copy(rows_v, out_hbm.at[pl.ds(base, b_per_w)])

    return k(table, idx)


def kernel(input, embd_weight, ln_gamma, ln_beta):
    bsz, seq, d = input.shape
    n = bsz * seq
    k = embd_weight.shape[0]
    x2 = input.reshape(n, d)
    wt = embd_weight
    g2 = ln_gamma.reshape(1, d)
    b2 = ln_beta.reshape(1, d)
    iota_f = jnp.arange(k, dtype=jnp.float32).reshape(1, k)
    idx = _ln_argmax(x2, wt, g2, b2, iota_f)
    q = _sc_gather(embd_weight, idx)
    return q.reshape(bsz, seq, d), idx.reshape(bsz, seq)


# speedup vs baseline: 1.3757x; 1.0174x over previous
import functools

import jax
import jax.numpy as jnp
from jax import lax
from jax.experimental import pallas as pl
from jax.experimental.pallas import tpu as pltpu
from jax.experimental.pallas import tpu_sc as plsc

_BM = 1024


def _ln_argmax_body(x_ref, wt_ref, g_ref, b_ref, iota_ref, idx_ref):
    x = x_ref[...]
    mu = jnp.mean(x, axis=-1, keepdims=True)
    var = jnp.mean((x - mu) ** 2, axis=-1, keepdims=True)
    h = (x - mu) / jnp.sqrt(var + 1e-5) * g_ref[...] + b_ref[...]
    logits = jnp.dot(h, wt_ref[...], preferred_element_type=jnp.float32)
    k = logits.shape[1]
    nrg = _BM // 128
    nch = k // 128
    lane_f = iota_ref[0:1, 0:128]
    big = jnp.float32(3.0e38)
    for g in range(nrg):
        rows = lax.slice(logits, (g * 128, 0), ((g + 1) * 128, k))
        m_run = lax.slice(rows, (0, 0), (128, 128))
        c_run = jnp.zeros((128, 128), jnp.float32)
        for c in range(1, nch):
            chunk = lax.slice(rows, (0, c * 128), (128, (c + 1) * 128))
            gt = chunk > m_run
            m_run = jnp.where(gt, chunk, m_run)
            c_run = jnp.where(gt, jnp.float32(c), c_run)
        m = jnp.max(m_run, axis=-1, keepdims=True)
        cand = jnp.where(m_run == m, c_run * 128.0 + lane_f, big)
        idxf = jnp.min(cand, axis=-1)
        idx_ref[g * (128 // 128):(g + 1) * (128 // 128), :] = (
            idxf.astype(jnp.int32).reshape(1, 128))


def _ln_argmax(x2, wt, g2, b2, iota_f):
    n, d = x2.shape
    k = wt.shape[1]
    grid = n // _BM
    rows = _BM // 128
    idx2 = pl.pallas_call(
        _ln_argmax_body,
        grid=(grid,),
        in_specs=[
            pl.BlockSpec((_BM, d), lambda i: (i, 0)),
            pl.BlockSpec((d, k), lambda i: (0, 0)),
            pl.BlockSpec((1, d), lambda i: (0, 0)),
            pl.BlockSpec((1, d), lambda i: (0, 0)),
            pl.BlockSpec((1, k), lambda i: (0, 0)),
        ],
        out_specs=pl.BlockSpec((rows, 128), lambda i: (i, 0)),
        out_shape=jax.ShapeDtypeStruct((grid * rows, 128), jnp.int32),
    )(x2, wt, g2, b2, iota_f)
    return idx2.reshape(n)


def _sc_gather(table, idx):
    b = idx.shape[0]
    d = table.shape[1]
    nw = 32
    b_per_w = b // nw
    mesh = plsc.VectorSubcoreMesh(core_axis_name="c", subcore_axis_name="s")

    @functools.partial(
        pl.kernel,
        mesh=mesh,
        compiler_params=pltpu.CompilerParams(use_tc_tiling_on_sc=False),
        out_type=jax.ShapeDtypeStruct((b, d), jnp.float32),
        scratch_types=[
            pltpu.VMEM((b_per_w,), jnp.int32),
            pltpu.VMEM((b_per_w, d), jnp.float32),
            pltpu.SemaphoreType.DMA,
        ],
    )
    def k(table_hbm, idx_hbm, out_hbm, idx_v, rows_v, sem):
        wid = lax.axis_index("s") * 2 + lax.axis_index("c")
        base = wid * b_per_w
        pltpu.sync_copy(idx_hbm.at[pl.ds(base, b_per_w)], idx_v)
        pltpu.async_copy(table_hbm.at[idx_v], rows_v, sem).wait()
        pltpu.sync_copy(rows_v, out_hbm.at[pl.ds(base, b_per_w)])

    return k(table, idx)


def kernel(input, embd_weight, ln_gamma, ln_beta):
    bsz, seq, d = input.shape
    n = bsz * seq
    k = embd_weight.shape[0]
    x2 = input.reshape(n, d)
    wt = embd_weight.T
    g2 = ln_gamma.reshape(1, d)
    b2 = ln_beta.reshape(1, d)
    iota_f = jnp.arange(k, dtype=jnp.float32).reshape(1, k)
    idx = _ln_argmax(x2, wt, g2, b2, iota_f)
    q = _sc_gather(embd_weight, idx)
    return q.reshape(bsz, seq, d), idx.reshape(bsz, seq)
